# Initial kernel scaffold; baseline (speedup 1.0000x reference)
import jax
import jax.numpy as jnp
from jax.experimental import pallas as pl


def kernel(feat, segment_ids, W, b):
    raise NotImplementedError("write your pallas kernel here")


# speedup vs baseline: 71.7657x; 71.7657x over previous
import functools

import jax
import jax.numpy as jnp
from jax.experimental import pallas as pl
from jax.experimental.pallas import tpu as pltpu

_NEG = -1e30


def _body(feat_ref, ids_ref, wt_ref, b_ref, out_ref, m_ref, d_ref, s_ref,
          *, nblocks, R, B, K, KP, D):
    i = pl.program_id(0)

    @pl.when(i == 0)
    def _init():
        m_ref[...] = jnp.full((B, KP), _NEG, jnp.float32)
        d_ref[...] = jnp.zeros((B, KP), jnp.float32)
        s_ref[...] = jnp.zeros((B, K, D), jnp.float32)

    feat = feat_ref[...]
    ids = ids_ref[0, 0, :]
    seg_iota = jax.lax.broadcasted_iota(jnp.int32, (R, B), 1)
    p_bool = ids[:, None] == seg_iota
    p = p_bool.astype(jnp.float32)

    g = jax.lax.dot_general(feat, wt_ref[...], (((1,), (0,)), ((), ())),
                            preferred_element_type=jnp.float32)
    g = g + b_ref[0:1, :]
    if KP > K:
        g = jnp.concatenate(
            [g[:, :K], jnp.full((R, KP - K), _NEG, jnp.float32)], axis=1)

    bms = []
    for k in range(K):
        masked = jnp.where(p_bool, g[:, k][:, None], _NEG)
        bms.append(jnp.max(masked, axis=0))
    bm = jnp.stack(bms, axis=1)
    if KP > K:
        bm = jnp.concatenate(
            [bm, jnp.full((B, KP - K), _NEG, jnp.float32)], axis=1)

    m_old = m_ref[...]
    m_new = jnp.maximum(m_old, bm)
    scale = jnp.exp(m_old - m_new)
    m_ref[...] = m_new

    m_row = jax.lax.dot_general(p, m_new, (((1,), (0,)), ((), ())),
                                preferred_element_type=jnp.float32)
    e = jnp.exp(g - m_row)

    dd = jax.lax.dot_general(p, e, (((0,), (0,)), ((), ())),
                             preferred_element_type=jnp.float32)
    d_ref[...] = d_ref[...] * scale + dd

    for k in range(K):
        pe = p * e[:, k][:, None]
        sk = jax.lax.dot_general(pe, feat, (((0,), (0,)), ((), ())),
                                 preferred_element_type=jnp.float32)
        s_ref[:, k, :] = s_ref[:, k, :] * scale[:, k][:, None] + sk

    @pl.when(i == nblocks - 1)
    def _final():
        d = d_ref[...][:, :K]
        denom = jnp.where(d > 0, d, jnp.float32(1.0))[:, :, None]
        out_ref[...] = jnp.where(d[:, :, None] > 0,
                                 s_ref[...] / denom,
                                 jnp.float32(0.0))


@jax.jit
def kernel(feat, segment_ids, W, b):
    N, D = feat.shape
    K = W.shape[0]
    B = 64
    KP = 8

    R = None
    for cand in (2000, 2500, 1000, 800, 500, 400, 250, 200, 125, 100, 8):
        if N % cand == 0 and cand % 8 == 0:
            R = cand
            break
    if R is None:
        R = 8
        pad = (-N) % R
        feat = jnp.pad(feat, ((0, pad), (0, 0)))
        segment_ids = jnp.pad(segment_ids, (0, pad), constant_values=B)
        N = N + pad
    nblocks = N // R

    ids3 = segment_ids.astype(jnp.int32).reshape(nblocks, 1, R)
    wt = jnp.zeros((D, KP), jnp.float32).at[:, :K].set(W.T.astype(jnp.float32))
    b8 = jnp.broadcast_to(
        jnp.zeros((KP,), jnp.float32).at[:K].set(b.astype(jnp.float32)),
        (8, KP))

    body = functools.partial(_body, nblocks=nblocks, R=R, B=B, K=K, KP=KP, D=D)
    out = pl.pallas_call(
        body,
        grid=(nblocks,),
        in_specs=[
            pl.BlockSpec((R, D), lambda i: (i, 0)),
            pl.BlockSpec((1, 1, R), lambda i: (i, 0, 0)),
            pl.BlockSpec((D, KP), lambda i: (0, 0)),
            pl.BlockSpec((8, KP), lambda i: (0, 0)),
        ],
        out_specs=pl.BlockSpec((B, K, D), lambda i: (0, 0, 0)),
        out_shape=jax.ShapeDtypeStruct((B, K, D), jnp.float32),
        scratch_shapes=[
            pltpu.VMEM((B, KP), jnp.float32),
            pltpu.VMEM((B, KP), jnp.float32),
            pltpu.VMEM((B, K, D), jnp.float32),
        ],
        compiler_params=pltpu.CompilerParams(
            dimension_semantics=("arbitrary",)),
    )(feat.astype(jnp.float32), ids3, wt, b8)
    return out
